# baseline (speedup 1.0000x reference)
import functools

import jax
import jax.numpy as jnp
from jax import lax
from jax.experimental import pallas as pl
from jax.experimental.pallas import tpu as pltpu
from jax.experimental.pallas import tpu_sc as plsc

_B = 64
_H = 384
_W = 384
_L = 16


def _sc_center_mse(pred_flat, target, cy, cx):
    mesh = plsc.VectorSubcoreMesh(core_axis_name="c", subcore_axis_name="s")

    @functools.partial(
        pl.kernel,
        mesh=mesh,
        out_type=jax.ShapeDtypeStruct((_L,), jnp.float32),
        scratch_types=[
            pltpu.VMEM((_B,), jnp.int32),
            pltpu.VMEM((_B,), jnp.float32),
            pltpu.VMEM((_B,), jnp.float32),
            pltpu.VMEM((_B,), jnp.int32),
            pltpu.VMEM((_B,), jnp.int32),
            pltpu.VMEM((_L,), jnp.float32),
            pltpu.SemaphoreType.DMA,
        ],
    )
    def k(pred_hbm, tgt_hbm, cy_hbm, cx_hbm, out_hbm,
          idx_v, val_v, tgt_v, cy_v, cx_v, res_v, sem):
        cid = lax.axis_index("c")
        sid = lax.axis_index("s")

        @pl.when(jnp.logical_and(cid == 0, sid == 0))
        def _():
            pltpu.sync_copy(cy_hbm, cy_v)
            pltpu.sync_copy(cx_hbm, cx_v)
            pltpu.sync_copy(tgt_hbm, tgt_v)
            for i in range(_B // _L):
                y = cy_v[pl.ds(i * _L, _L)]
                x = cx_v[pl.ds(i * _L, _L)]
                b = lax.iota(jnp.int32, _L) + (i * _L)
                idx_v[pl.ds(i * _L, _L)] = b * (_H * _W) + y * _W + x
            pltpu.async_copy(pred_hbm.at[idx_v], val_v, sem).wait()
            acc = jnp.zeros((_L,), jnp.float32)
            for i in range(_B // _L):
                d = val_v[pl.ds(i * _L, _L)] - tgt_v[pl.ds(i * _L, _L)]
                acc = acc + d * d
            total = acc[0]
            for j in range(1, _L):
                total = total + acc[j]
            total = total * (1.0 / _B)
            res_v[...] = total * jnp.ones((_L,), jnp.float32)
            pltpu.sync_copy(res_v, out_hbm)

    return k(pred_flat, target, cy, cx)


def kernel(pred, target, center_yx):
    pred_flat = pred.reshape(_B * _H * _W)
    yx = center_yx.astype(jnp.int32)
    out = _sc_center_mse(pred_flat, target, yx[:, 0], yx[:, 1])
    return out[0]


# speedup vs baseline: 9.7302x; 9.7302x over previous
import jax
import jax.numpy as jnp
from jax.experimental import pallas as pl
from jax.experimental.pallas import tpu as pltpu

_B = 64
_H = 384
_W = 384


def _body(pred_ref, yx_s, yx_v, tgt_ref, out_ref, rows_ref, sem):
    copies = []
    for b in range(_B):
        cy = yx_s[b, 0]
        c = pltpu.make_async_copy(
            pred_ref.at[b, 0, cy, :], rows_ref.at[b], sem)
        c.start()
        copies.append(c)
    for c in copies:
        c.wait()
    cx = yx_v[:, 1:2]
    lane = jax.lax.broadcasted_iota(jnp.int32, (_B, _W), 1)
    d = jnp.where(lane == cx, rows_ref[...] - tgt_ref[...], 0.0)
    out_ref[0] = jnp.sum(d * d) * (1.0 / _B)


def kernel(pred, target, center_yx):
    yx = center_yx.astype(jnp.int32)
    tgt = target.reshape(_B, 1)
    out = pl.pallas_call(
        _body,
        out_shape=jax.ShapeDtypeStruct((1,), jnp.float32),
        in_specs=[
            pl.BlockSpec(memory_space=pl.ANY),
            pl.BlockSpec(memory_space=pltpu.SMEM),
            pl.BlockSpec(memory_space=pltpu.VMEM),
            pl.BlockSpec(memory_space=pltpu.VMEM),
        ],
        out_specs=pl.BlockSpec(memory_space=pltpu.SMEM),
        scratch_shapes=[
            pltpu.VMEM((_B, _W), jnp.float32),
            pltpu.SemaphoreType.DMA,
        ],
    )(pred, yx, yx, tgt)
    return out[0]


# speedup vs baseline: 12.2310x; 1.2570x over previous
import jax
import jax.numpy as jnp
from jax.experimental import pallas as pl
from jax.experimental.pallas import tpu as pltpu

_B = 64
_H = 384
_W = 384


def _body(pred_ref, yx_s, yx_v, tgt_ref, out_ref, rows_ref, sem):
    copies = []
    for b in range(_B):
        cy = yx_s[b, 0]
        c = pltpu.make_async_copy(
            pred_ref.at[b, 0, cy, :], rows_ref.at[b], sem)
        c.start()
        copies.append(c)
    for c in copies:
        c.wait()
    cx = yx_v[:, 1:2]
    lane = jax.lax.broadcasted_iota(jnp.int32, (_B, _W), 1)
    g = jnp.sum(jnp.where(lane == cx, rows_ref[...], 0.0), axis=1)
    d = g - tgt_ref[...]
    out_ref[0] = jnp.sum(d * d) * (1.0 / _B)


def kernel(pred, target, center_yx):
    yx = center_yx.astype(jnp.int32)
    out = pl.pallas_call(
        _body,
        out_shape=jax.ShapeDtypeStruct((1,), jnp.float32),
        in_specs=[
            pl.BlockSpec(memory_space=pl.ANY),
            pl.BlockSpec(memory_space=pltpu.SMEM),
            pl.BlockSpec(memory_space=pltpu.VMEM),
            pl.BlockSpec(memory_space=pltpu.VMEM),
        ],
        out_specs=pl.BlockSpec(memory_space=pltpu.SMEM),
        scratch_shapes=[
            pltpu.VMEM((_B, _W), jnp.float32),
            pltpu.SemaphoreType.DMA,
        ],
    )(pred, yx, yx, target)
    return out[0]


# speedup vs baseline: 12.2621x; 1.0025x over previous
import jax
import jax.numpy as jnp
from jax.experimental import pallas as pl
from jax.experimental.pallas import tpu as pltpu

_B = 64
_H = 384
_W = 384


def _body(pred_ref, yx_s, yx_v, tgt_ref, out_ref, rows_ref, sem):
    copies = []
    for b in range(_B):
        cy = yx_s[b, 0]
        cx0 = pl.multiple_of(yx_s[b, 1] & ~127, 128)
        c = pltpu.make_async_copy(
            pred_ref.at[b, 0, cy, pl.ds(cx0, 128)], rows_ref.at[b], sem)
        c.start()
        copies.append(c)
    for c in copies:
        c.wait()
    cx = yx_v[:, 1:2] & 127
    lane = jax.lax.broadcasted_iota(jnp.int32, (_B, 128), 1)
    g = jnp.sum(jnp.where(lane == cx, rows_ref[...], 0.0), axis=1)
    d = g - tgt_ref[...]
    out_ref[0] = jnp.sum(d * d) * (1.0 / _B)


def kernel(pred, target, center_yx):
    yx = center_yx.astype(jnp.int32)
    out = pl.pallas_call(
        _body,
        out_shape=jax.ShapeDtypeStruct((1,), jnp.float32),
        in_specs=[
            pl.BlockSpec(memory_space=pl.ANY),
            pl.BlockSpec(memory_space=pltpu.SMEM),
            pl.BlockSpec(memory_space=pltpu.VMEM),
            pl.BlockSpec(memory_space=pltpu.VMEM),
        ],
        out_specs=pl.BlockSpec(memory_space=pltpu.SMEM),
        scratch_shapes=[
            pltpu.VMEM((_B, 128), jnp.float32),
            pltpu.SemaphoreType.DMA,
        ],
    )(pred, yx, yx, target)
    return out[0]
